# baseline (speedup 1.0000x reference)
import functools

import jax
import jax.numpy as jnp
from jax import lax
from jax.experimental import pallas as pl
from jax.experimental.pallas import tpu as pltpu
from jax.experimental.pallas import tpu_sc as plsc

_LANES = 16
_NC = 2
_NS = 16
_DT = 2


@functools.partial(jax.jit, static_argnums=(2, 3, 4, 5))
def _select_t(x, indices, B, T, C, K):
    n_workers = _NC * _NS
    bw = B // n_workers
    dt = _DT
    n_steps = T // dt
    rows = dt * bw
    n_groups = bw // _LANES

    mesh = plsc.VectorSubcoreMesh(
        core_axis_name="c", subcore_axis_name="s",
        num_cores=_NC, num_subcores=_NS)

    @functools.partial(
        pl.kernel,
        out_type=jax.ShapeDtypeStruct((T, K, B), jnp.float32),
        mesh=mesh,
        scratch_types=[
            pltpu.VMEM((K,), jnp.int32),
            pltpu.VMEM((2, dt, bw), jnp.int32),
            pltpu.VMEM((2, dt, bw, C), jnp.float32),
            pltpu.VMEM((2, dt, K, bw), jnp.float32),
            pltpu.SemaphoreType.DMA,
            pltpu.SemaphoreType.DMA,
            pltpu.SemaphoreType.DMA,
            pltpu.SemaphoreType.DMA,
        ],
        compiler_params=pltpu.CompilerParams(needs_layout_passes=False),
    )
    def body(x_hbm, idx_hbm, out_hbm, idx_v, rid_v, in_v, out_v,
             sin0, sin1, sout0, sout1):
        sin = (sin0, sin1)
        sout = (sout0, sout1)
        wid = lax.axis_index("s") * _NC + lax.axis_index("c")
        b0 = wid * bw
        pltpu.sync_copy(idx_hbm, idx_v)

        lane = lax.iota(jnp.int32, _LANES)
        mvecs = [(lane + g * _LANES) * T for g in range(n_groups)]
        rvecs = [lane + g * _LANES for g in range(n_groups)]

        def start_in(i, s):
            t0 = i * dt
            for u in range(dt):
                for g in range(n_groups):
                    rid_v[s, u, pl.ds(g * _LANES, _LANES)] = (
                        mvecs[g] + (b0 * T + t0 + u))
            for u in range(dt):
                pltpu.async_copy(x_hbm.at[rid_v.at[s, u]], in_v.at[s, u],
                                 sin[s])

        def wait_in(i, s):
            for u in range(dt):
                pltpu.make_async_copy(x_hbm.at[rid_v.at[s, u]],
                                      in_v.at[s, u], sin[s]).wait()

        def out_slice(i):
            return out_hbm.at[pl.ds(i * dt, dt), :, pl.ds(b0, bw)]

        def start_out(i, s):
            pltpu.async_copy(out_v.at[s], out_slice(i), sout[s])

        def wait_out(i, s):
            pltpu.make_async_copy(out_v.at[s], out_slice(i), sout[s]).wait()

        def compute(s):
            for u in range(dt):
                @plsc.parallel_loop(0, K, unroll=2)
                def _(j):
                    jv = jnp.full((_LANES,), j, jnp.int32)
                    cv = plsc.load_gather(idx_v, [jv])
                    for g in range(n_groups):
                        out_v[s, u, j, pl.ds(g * _LANES, _LANES)] = (
                            plsc.load_gather(in_v.at[s, u], [rvecs[g], cv]))

        start_in(0, 0)
        start_in(1, 1)
        for s in (0, 1):
            wait_in(s, s)
            compute(s)
            start_out(s, s)
            start_in(s + 2, s)

        def loop_body(i2, carry):
            for s in (0, 1):
                i = 2 * i2 + s
                wait_in(i, s)
                wait_out(i - 2, s)
                compute(s)
                start_out(i, s)

                @pl.when(i2 < n_steps // 2 - 1)
                def _():
                    start_in(i + 2, s)

            return carry

        lax.fori_loop(1, n_steps // 2, loop_body, 0)

        wait_out(n_steps - 2, 0)
        wait_out(n_steps - 1, 1)

    return body(x, indices)


def kernel(x, indices):
    B, T, C = x.shape
    K = indices.shape[0]
    out_t = _select_t(x.reshape(B * T, C), indices.astype(jnp.int32),
                      B, T, C, K)
    return jnp.transpose(out_t, (2, 0, 1))
